# baseline (speedup 1.0000x reference)
import functools
import jax
import jax.numpy as jnp
from jax.experimental import pallas as pl
from jax.experimental.pallas import tpu as pltpu

N = 10000
E = 160000
NPAD = 10048
EPAD = 163840


def _mm1_body(a_ref, w_ref, b_ref, *out_refs):
    a = a_ref[...].astype(jnp.bfloat16)
    for i, o in enumerate(out_refs):
        o[...] = (
            jnp.dot(a, w_ref[i], preferred_element_type=jnp.float32) + b_ref[i]
        )


def _mm1(ea, w5, b5):
    R = 1280
    grid = (EPAD // R,)
    nlast = E // R - 1
    return pl.pallas_call(
        _mm1_body,
        grid=grid,
        in_specs=[
            pl.BlockSpec((R, 345), lambda i: (jnp.minimum(i, nlast), 0)),
            pl.BlockSpec((5, 345, 64), lambda i: (0, 0, 0)),
            pl.BlockSpec((5, 64), lambda i: (0, 0)),
        ],
        out_specs=[pl.BlockSpec((R, 64), lambda i: (i, 0))] * 5,
        out_shape=[jax.ShapeDtypeStruct((EPAD, 64), jnp.float32)] * 5,
    )(ea, w5, b5)


def _mm2_body(a_ref, w_ref, b_ref, o_ref):
    o_ref[...] = (
        jnp.dot(a_ref[...], w_ref[...], preferred_element_type=jnp.float32)
        + b_ref[...]
    )


def _mm2(a, w, b):
    R = 2048
    return pl.pallas_call(
        _mm2_body,
        grid=(EPAD // R,),
        in_specs=[
            pl.BlockSpec((R, 64), lambda i: (i, 0)),
            pl.BlockSpec((64, 64), lambda i: (0, 0)),
            pl.BlockSpec((1, 64), lambda i: (0, 0)),
        ],
        out_specs=pl.BlockSpec((R, 64), lambda i: (i, 0)),
        out_shape=jax.ShapeDtypeStruct((EPAD, 64), jnp.float32),
    )(a, w, b)


def _proj_body(xc_ref, ws_ref, wd_ref, a_out, b_out):
    xc = xc_ref[...]
    a_out[...] = jnp.zeros((NPAD, 64), jnp.float32)
    b_out[...] = jnp.zeros((NPAD, 64), jnp.float32)
    a_out[:N, :] = jnp.dot(xc, ws_ref[...], preferred_element_type=jnp.float32)
    b_out[:N, :] = jnp.dot(xc, wd_ref[...], preferred_element_type=jnp.float32)


def _proj(xc, ws, wd):
    return pl.pallas_call(
        _proj_body,
        out_shape=[jax.ShapeDtypeStruct((NPAD, 64), jnp.float32)] * 2,
    )(xc, ws, wd)


def _nd_body(h_ref, wn_ref, bn_ref, a0_ref, a1_ref, g_ref, bt_ref,
             ws_ref, wd_ref, h_out, a_out, b_out):
    y = jnp.dot(h_ref[...], wn_ref[...], preferred_element_type=jnp.float32)
    y = y + bn_ref[...] + a0_ref[:N, :50] + a1_ref[:N, :50]
    mu = jnp.mean(y, axis=0, keepdims=True)
    var = jnp.mean((y - mu) ** 2, axis=0, keepdims=True)
    yn = g_ref[...] * (y - mu) * jax.lax.rsqrt(var + 1e-5) + bt_ref[...]
    h = jnp.where(yn > 0, yn, jnp.exp(jnp.minimum(yn, 0.0)) - 1.0)
    h_out[...] = h
    a_out[...] = jnp.zeros((NPAD, 64), jnp.float32)
    b_out[...] = jnp.zeros((NPAD, 64), jnp.float32)
    a_out[:N, :] = jnp.dot(h, ws_ref[...], preferred_element_type=jnp.float32)
    b_out[:N, :] = jnp.dot(h, wd_ref[...], preferred_element_type=jnp.float32)


def _node_dense(h, wn, bn, a0, a1, g, bt, ws, wd):
    return pl.pallas_call(
        _nd_body,
        out_shape=[
            jax.ShapeDtypeStruct((N, 50), jnp.float32),
            jax.ShapeDtypeStruct((NPAD, 64), jnp.float32),
            jax.ShapeDtypeStruct((NPAD, 64), jnp.float32),
        ],
    )(h, wn, bn, a0, a1, g, bt, ws, wd)


def _ndf_body(h_ref, wn_ref, bn_ref, a0_ref, a1_ref, g_ref, bt_ref, h_out):
    y = jnp.dot(h_ref[...], wn_ref[...], preferred_element_type=jnp.float32)
    y = y + bn_ref[...] + a0_ref[:N, :50] + a1_ref[:N, :50]
    mu = jnp.mean(y, axis=0, keepdims=True)
    var = jnp.mean((y - mu) ** 2, axis=0, keepdims=True)
    yn = g_ref[...] * (y - mu) * jax.lax.rsqrt(var + 1e-5) + bt_ref[...]
    h_out[...] = jnp.where(yn > 0, yn, jnp.exp(jnp.minimum(yn, 0.0)) - 1.0)


def _node_final(h, wn, bn, a0, a1, g, bt):
    return pl.pallas_call(
        _ndf_body,
        out_shape=jax.ShapeDtypeStruct((N, 50), jnp.float32),
    )(h, wn, bn, a0, a1, g, bt)


def _head_body(o_ref, w_ref, b_ref, g_ref, bt_ref, out_ref):
    y = jnp.dot(o_ref[...], w_ref[...], preferred_element_type=jnp.float32)
    y = y + b_ref[...]
    mu = jnp.mean(y, axis=0, keepdims=True)
    var = jnp.mean((y - mu) ** 2, axis=0, keepdims=True)
    out_ref[...] = g_ref[...] * (y - mu) * jax.lax.rsqrt(var + 1e-5) + bt_ref[...]


def _head(o, w, b, g, bt):
    return pl.pallas_call(
        _head_body,
        out_shape=jax.ShapeDtypeStruct((N, 196), jnp.float32),
    )(o, w, b, g, bt)


def _msg_pass(a, b, c, src, dst, need_eout):
    e = jax.nn.relu(a[src] + b[dst] + c[:E, :])
    agg = jax.ops.segment_sum(e, dst, num_segments=NPAD)
    if need_eout:
        ep = jnp.zeros((EPAD, 64), jnp.float32).at[:E].set(e)
    else:
        ep = None
    return ep, agg, jnp.zeros((NPAD, 64), jnp.float32)


def _pad_w(w, rows, cols):
    out = jnp.zeros((rows, cols), jnp.float32)
    return out.at[: w.shape[0], : w.shape[1]].set(w)


def kernel(x, edge_index, edge_attr, condition, params):
    xc = jnp.concatenate([x, condition], axis=1)
    src = edge_index[0]
    dst = edge_index[1]

    w5 = jnp.stack(
        [
            _pad_w(params["conv%d_1" % b]["We"][92:], 345, 64)
            for b in range(1, 6)
        ]
    ).astype(jnp.bfloat16)
    b5 = jnp.stack(
        [_pad_w(params["conv%d_1" % b]["be"][None, :], 1, 64)[0] for b in range(1, 6)]
    )
    c1 = _mm1(edge_attr, w5, b5)

    outs = []
    for b in range(1, 6):
        hb = xc
        c = c1[b - 1]
        for i in range(1, b + 1):
            p = params["conv%d_%d" % (b, i)]
            in_c = p["Wn"].shape[0]
            ws = _pad_w(p["We"][:in_c], in_c, 64)
            wd = _pad_w(p["We"][in_c : 2 * in_c], in_c, 64)
            if i == 1:
                a_nodes, b_nodes = _proj(xc, ws, wd)
            need_eout = i < b
            e_out, agg0, agg1 = _msg_pass(a_nodes, b_nodes, c, src, dst, need_eout)
            bnp = params["BN%d_%d" % (b, i)]
            if i < b:
                pn = params["conv%d_%d" % (b, i + 1)]
                wsn = _pad_w(pn["We"][:50], 50, 64)
                wdn = _pad_w(pn["We"][50:100], 50, 64)
                hb, a_nodes, b_nodes = _node_dense(
                    hb, p["Wn"], p["bn"][None, :], agg0, agg1,
                    bnp["gamma"][None, :], bnp["beta"][None, :], wsn, wdn,
                )
                wen = _pad_w(pn["We"][100:], 64, 64)
                ben = _pad_w(pn["be"][None, :], 1, 64)
                c = _mm2(e_out, wen, ben)
            else:
                hb = _node_final(
                    hb, p["Wn"], p["bn"][None, :], agg0, agg1,
                    bnp["gamma"][None, :], bnp["beta"][None, :],
                )
        outs.append(hb)

    o = jnp.concatenate(outs, axis=1)
    o2 = _head(
        o, params["linR_W"], params["linR_b"][None, :],
        params["BNR"]["gamma"][None, :], params["BNR"]["beta"][None, :],
    )
    return o2.reshape(-1, 49, 4)


# speedup vs baseline: 1.3162x; 1.3162x over previous
import functools
import jax
import jax.numpy as jnp
from jax import lax
from jax.experimental import pallas as pl
from jax.experimental.pallas import tpu as pltpu
from jax.experimental.pallas import tpu_sc as plsc

N = 10000
E = 160000
NPAD = 10112
F = 128
H = 5120
AGGR = 5248
E2 = 86016
NW = 32
STEPS = 21
BLK = 128
ARS = AGGR // 16

_PACKS = {
    1: [(1, 2), (3, 4), (5,)],
    2: [(2, 3), (4, 5)],
    3: [(3, 4), (5,)],
    4: [(4, 5)],
    5: [(5,)],
}


def _mm1_body(a_ref, w_ref, b_ref, *out_refs):
    a = a_ref[...].astype(jnp.bfloat16)
    for i, o in enumerate(out_refs):
        o[...] = (
            jnp.dot(a, w_ref[i], preferred_element_type=jnp.float32) + b_ref[i]
        )


def _mm1(ea, w3, b3):
    R = 1280
    return pl.pallas_call(
        _mm1_body,
        grid=(E // R,),
        in_specs=[
            pl.BlockSpec((R, 345), lambda i: (i, 0)),
            pl.BlockSpec((3, 345, F), lambda i: (0, 0, 0)),
            pl.BlockSpec((3, F), lambda i: (0, 0)),
        ],
        out_specs=[pl.BlockSpec((R, F), lambda i: (i, 0))] * 3,
        out_shape=[jax.ShapeDtypeStruct((E, F), jnp.float32)] * 3,
    )(ea, w3, b3)


def _mm2_pair_body(off_l, off_r, el_ref, er_ref, wl_ref, wr_ref,
                   bl_ref, br_ref, o_ref):
    cl = jnp.dot(el_ref[...][:, off_l:off_l + 64], wl_ref[...],
                 preferred_element_type=jnp.float32) + bl_ref[...]
    cr = jnp.dot(er_ref[...][:, off_r:off_r + 64], wr_ref[...],
                 preferred_element_type=jnp.float32) + br_ref[...]
    o_ref[...] = jnp.concatenate([cl, cr], axis=1)


def _mm2_pair(el, er, wl, wr, bl, br, off_l, off_r):
    R = 2048
    return pl.pallas_call(
        functools.partial(_mm2_pair_body, off_l, off_r),
        grid=(E2 // R,),
        in_specs=[
            pl.BlockSpec((R, F), lambda i: (i, 0)),
            pl.BlockSpec((R, F), lambda i: (i, 0)),
            pl.BlockSpec((64, 64), lambda i: (0, 0)),
            pl.BlockSpec((64, 64), lambda i: (0, 0)),
            pl.BlockSpec((1, 64), lambda i: (0, 0)),
            pl.BlockSpec((1, 64), lambda i: (0, 0)),
        ],
        out_specs=pl.BlockSpec((R, F), lambda i: (i, 0)),
        out_shape=jax.ShapeDtypeStruct((E2, F), jnp.float32),
    )(el, er, wl, wr, bl, br)


def _mm2_single_body(off_l, el_ref, wl_ref, bl_ref, o_ref):
    cl = jnp.dot(el_ref[...][:, off_l:off_l + 64], wl_ref[...],
                 preferred_element_type=jnp.float32) + bl_ref[...]
    o_ref[...] = jnp.concatenate([cl, jnp.zeros_like(cl)], axis=1)


def _mm2_single(el, wl, bl, off_l):
    R = 2048
    return pl.pallas_call(
        functools.partial(_mm2_single_body, off_l),
        grid=(E2 // R,),
        in_specs=[
            pl.BlockSpec((R, F), lambda i: (i, 0)),
            pl.BlockSpec((64, 64), lambda i: (0, 0)),
            pl.BlockSpec((1, 64), lambda i: (0, 0)),
        ],
        out_specs=pl.BlockSpec((R, F), lambda i: (i, 0)),
        out_shape=jax.ShapeDtypeStruct((E2, F), jnp.float32),
    )(el, wl, bl)


def _proj_pair_body(hl_ref, hr_ref, wsl_ref, wdl_ref, wsr_ref, wdr_ref,
                    a_out, b_out):
    hl = hl_ref[...]
    hr = hr_ref[...]
    a_out[...] = jnp.zeros((NPAD, F), jnp.float32)
    b_out[...] = jnp.zeros((NPAD, F), jnp.float32)
    a_out[:N, :] = jnp.concatenate(
        [jnp.dot(hl, wsl_ref[...], preferred_element_type=jnp.float32),
         jnp.dot(hr, wsr_ref[...], preferred_element_type=jnp.float32)], axis=1)
    b_out[:N, :] = jnp.concatenate(
        [jnp.dot(hl, wdl_ref[...], preferred_element_type=jnp.float32),
         jnp.dot(hr, wdr_ref[...], preferred_element_type=jnp.float32)], axis=1)


def _proj_pair(hl, hr, wsl, wdl, wsr, wdr):
    return pl.pallas_call(
        _proj_pair_body,
        out_shape=[jax.ShapeDtypeStruct((NPAD, F), jnp.float32)] * 2,
    )(hl, hr, wsl, wdl, wsr, wdr)


def _proj_single_body(hl_ref, wsl_ref, wdl_ref, a_out, b_out):
    hl = hl_ref[...]
    a_out[...] = jnp.zeros((NPAD, F), jnp.float32)
    b_out[...] = jnp.zeros((NPAD, F), jnp.float32)
    al = jnp.dot(hl, wsl_ref[...], preferred_element_type=jnp.float32)
    bl = jnp.dot(hl, wdl_ref[...], preferred_element_type=jnp.float32)
    a_out[:N, :] = jnp.concatenate([al, jnp.zeros_like(al)], axis=1)
    b_out[:N, :] = jnp.concatenate([bl, jnp.zeros_like(bl)], axis=1)


def _proj_single(hl, wsl, wdl):
    return pl.pallas_call(
        _proj_single_body,
        out_shape=[jax.ShapeDtypeStruct((NPAD, F), jnp.float32)] * 2,
    )(hl, wsl, wdl)


def _nd_body(off, h_ref, wn_ref, bn_ref, aga_ref, agb_ref, g_ref, bt_ref,
             h_out):
    y = jnp.dot(h_ref[...], wn_ref[...], preferred_element_type=jnp.float32)
    agg = jnp.concatenate(
        [aga_ref[0, :H, off:off + 50] + aga_ref[1, :H, off:off + 50],
         agb_ref[0, :N - H, off:off + 50] + agb_ref[1, :N - H, off:off + 50]],
        axis=0)
    y = y + bn_ref[...] + agg
    mu = jnp.mean(y, axis=0, keepdims=True)
    var = jnp.mean((y - mu) ** 2, axis=0, keepdims=True)
    yn = g_ref[...] * (y - mu) * jax.lax.rsqrt(var + 1e-5) + bt_ref[...]
    h_out[...] = jnp.where(yn > 0, yn, jnp.exp(jnp.minimum(yn, 0.0)) - 1.0)


def _node_upd(h, wn, bn, aga, agb, g, bt, off):
    return pl.pallas_call(
        functools.partial(_nd_body, off),
        out_shape=jax.ShapeDtypeStruct((N, 50), jnp.float32),
    )(h, wn, bn, aga, agb, g, bt)


def _head_body(o_ref, w_ref, b_ref, g_ref, bt_ref, out_ref):
    y = jnp.dot(o_ref[...], w_ref[...], preferred_element_type=jnp.float32)
    y = y + b_ref[...]
    mu = jnp.mean(y, axis=0, keepdims=True)
    var = jnp.mean((y - mu) ** 2, axis=0, keepdims=True)
    out_ref[...] = g_ref[...] * (y - mu) * jax.lax.rsqrt(var + 1e-5) + bt_ref[...]


def _head(o, w, b, g, bt):
    return pl.pallas_call(
        _head_body,
        out_shape=jax.ShapeDtypeStruct((N, 196), jnp.float32),
    )(o, w, b, g, bt)


def _sc_body(layer1, a_hbm, b_hbm, c_hbm, si_hbm, di_hbm, dl_hbm, *rest):
    if layer1:
        (ci_hbm, eout_hbm, agg_hbm, idx_s, idx_d, idx_l, idx_c,
         buf_a, buf_b, buf_c, agg_sh, sem_a, sem_b, sem_c) = rest
    else:
        (eout_hbm, agg_hbm, idx_s, idx_d, idx_l, idx_c,
         buf_a, buf_b, buf_c, agg_sh, sem_a, sem_b, sem_c) = rest
        ci_hbm = None
    cid = lax.axis_index("c")
    sid = lax.axis_index("s")
    wid = cid * 16 + sid

    def zrow(r, c2):
        for q in range(F // 16):
            buf_a[r, pl.ds(q * 16, 16)] = jnp.zeros((16,), jnp.float32)
        return c2
    lax.fori_loop(0, BLK, zrow, 0)
    for k in range(2):
        pltpu.sync_copy(buf_a, agg_sh.at[pl.ds(sid * ARS + k * BLK, BLK)])
    pltpu.sync_copy(buf_a.at[pl.ds(0, ARS - 2 * BLK)],
                    agg_sh.at[pl.ds(sid * ARS + 2 * BLK, ARS - 2 * BLK)])
    pltpu.sync_copy(si_hbm.at[wid], idx_s)
    pltpu.sync_copy(di_hbm.at[wid], idx_d)
    pltpu.sync_copy(dl_hbm.at[wid], idx_l)
    if layer1:
        pltpu.sync_copy(ci_hbm.at[wid], idx_c)
    plsc.subcore_barrier()

    def step(j, carry):
        base = wid * (STEPS * BLK) + j * BLK
        cp_a = pltpu.async_copy(a_hbm.at[idx_s.at[j]], buf_a, sem_a)
        cp_b = pltpu.async_copy(b_hbm.at[idx_d.at[j]], buf_b, sem_b)
        if layer1:
            cp_c = pltpu.async_copy(c_hbm.at[idx_c.at[j]], buf_c, sem_c)
            cp_c.wait()
        else:
            pltpu.sync_copy(c_hbm.at[pl.ds(base, BLK)], buf_c)
        cp_a.wait()
        cp_b.wait()

        def row(r, c2):
            for q in range(F // 16):
                v = (buf_a[r, pl.ds(q * 16, 16)]
                     + buf_b[r, pl.ds(q * 16, 16)]
                     + buf_c[r, pl.ds(q * 16, 16)])
                buf_c[r, pl.ds(q * 16, 16)] = jnp.maximum(v, 0.0)
            return c2

        lax.fori_loop(0, BLK, row, 0)
        pltpu.sync_copy(buf_c, eout_hbm.at[pl.ds(base, BLK)])
        pltpu.sync_copy(buf_c, agg_sh.at[idx_l.at[j]], add=True)
        return carry

    lax.fori_loop(0, STEPS, step, 0)
    plsc.subcore_barrier()
    pltpu.sync_copy(agg_sh.at[pl.ds(sid * ARS, ARS)],
                    agg_hbm.at[cid, pl.ds(sid * ARS, ARS)])


def _make_sc(layer1):
    scratch = [
        pltpu.VMEM((STEPS, BLK), jnp.int32),
        pltpu.VMEM((STEPS, BLK), jnp.int32),
        pltpu.VMEM((STEPS, BLK), jnp.int32),
        pltpu.VMEM((STEPS, BLK), jnp.int32),
        pltpu.VMEM((BLK, F), jnp.float32),
        pltpu.VMEM((BLK, F), jnp.float32),
        pltpu.VMEM((BLK, F), jnp.float32),
        pltpu.VMEM_SHARED((AGGR, F), jnp.float32),
        pltpu.SemaphoreType.DMA,
        pltpu.SemaphoreType.DMA,
        pltpu.SemaphoreType.DMA,
    ]
    return pl.kernel(
        functools.partial(_sc_body, layer1),
        out_type=[
            jax.ShapeDtypeStruct((E2, F), jnp.float32),
            jax.ShapeDtypeStruct((2, AGGR, F), jnp.float32),
        ],
        mesh=plsc.VectorSubcoreMesh(core_axis_name="c", subcore_axis_name="s"),
        scratch_types=scratch,
    )


_SC_CONV1 = _make_sc(True)
_SC_CONVN = _make_sc(False)


def _pad_w(w, rows, cols):
    out = jnp.zeros((rows, cols), jnp.float32)
    return out.at[: w.shape[0], : w.shape[1]].set(w)


def _partition(src, dst):
    eid = jnp.arange(E, dtype=jnp.int32)
    srcx = jnp.concatenate([src, jnp.array([N], jnp.int32)])
    dstx = jnp.concatenate([dst, jnp.array([N], jnp.int32)])
    halves = []
    for h in range(2):
        mask = (dst >= h * H) & (dst < (h + 1) * H)
        pos = jnp.where(mask, jnp.cumsum(mask) - 1, E2)
        slots = jnp.full((E2,), E, jnp.int32).at[pos].set(eid, mode="drop")
        s = srcx[slots]
        d = dstx[slots]
        dl = jnp.clip(d - h * H, 0, AGGR - 2)
        cidx = jnp.minimum(slots, E - 1)
        halves.append(tuple(a.reshape(NW, STEPS, BLK) for a in (s, d, dl, cidx)))
    return halves


def kernel(x, edge_index, edge_attr, condition, params):
    xc = jnp.concatenate([x, condition], axis=1)
    src = edge_index[0]
    dst = edge_index[1]
    halves = _partition(src, dst)

    def w1_half(b):
        return _pad_w(params["conv%d_1" % b]["We"][92:], 345, 64)

    def b1_half(b):
        return _pad_w(params["conv%d_1" % b]["be"][None, :], 1, 64)[0]

    zw = jnp.zeros((345, 64), jnp.float32)
    zb = jnp.zeros((64,), jnp.float32)
    w3 = jnp.stack([
        jnp.concatenate([w1_half(1), w1_half(2)], axis=1),
        jnp.concatenate([w1_half(3), w1_half(4)], axis=1),
        jnp.concatenate([w1_half(5), zw], axis=1),
    ]).astype(jnp.bfloat16)
    b3 = jnp.stack([
        jnp.concatenate([b1_half(1), b1_half(2)]),
        jnp.concatenate([b1_half(3), b1_half(4)]),
        jnp.concatenate([b1_half(5), zb]),
    ])
    c1 = _mm1(edge_attr, w3, b3)

    h = {b: xc for b in range(1, 6)}
    esrc = {}
    for pos in range(1, 6):
        for ip, pack in enumerate(_PACKS[pos]):
            convs = [params["conv%d_%d" % (b, pos)] for b in pack]
            in_c = convs[0]["Wn"].shape[0]
            ws = [_pad_w(p["We"][:in_c], in_c, 64) for p in convs]
            wd = [_pad_w(p["We"][in_c:2 * in_c], in_c, 64) for p in convs]
            if len(pack) == 2:
                a_nodes, b_nodes = _proj_pair(
                    h[pack[0]], h[pack[1]], ws[0], wd[0], ws[1], wd[1])
            else:
                a_nodes, b_nodes = _proj_single(h[pack[0]], ws[0], wd[0])
            if pos > 1:
                we = [_pad_w(p["We"][2 * in_c:], 64, 64) for p in convs]
                be = [_pad_w(p["be"][None, :], 1, 64) for p in convs]
            eo, ag = [], []
            for hi in range(2):
                si, di, dl, ci = halves[hi]
                if pos == 1:
                    e_h, agg_h = _SC_CONV1(a_nodes, b_nodes, c1[ip],
                                           si, di, dl, ci)
                else:
                    if len(pack) == 2:
                        (el_ab, offl) = esrc[pack[0]]
                        (er_ab, offr) = esrc[pack[1]]
                        c_h = _mm2_pair(el_ab[hi], er_ab[hi], we[0], we[1],
                                        be[0], be[1], offl, offr)
                    else:
                        (el_ab, offl) = esrc[pack[0]]
                        c_h = _mm2_single(el_ab[hi], we[0], be[0], offl)
                    e_h, agg_h = _SC_CONVN(a_nodes, b_nodes, c_h, si, di, dl)
                eo.append(e_h)
                ag.append(agg_h)
            for k, b in enumerate(pack):
                p = convs[k]
                bnp = params["BN%d_%d" % (b, pos)]
                h[b] = _node_upd(h[b], p["Wn"], p["bn"][None, :],
                                 ag[0], ag[1],
                                 bnp["gamma"][None, :], bnp["beta"][None, :],
                                 64 * k)
                esrc[b] = ((eo[0], eo[1]), 64 * k)

    o = jnp.concatenate([h[b] for b in range(1, 6)], axis=1)
    o2 = _head(
        o, params["linR_W"], params["linR_b"][None, :],
        params["BNR"]["gamma"][None, :], params["BNR"]["beta"][None, :],
    )
    return o2.reshape(-1, 49, 4)


# speedup vs baseline: 1.3541x; 1.0288x over previous
import functools
import jax
import jax.numpy as jnp
from jax import lax
from jax.experimental import pallas as pl
from jax.experimental.pallas import tpu as pltpu
from jax.experimental.pallas import tpu_sc as plsc

N = 10000
E = 160000
NPAD = 10112
F = 128
H = 5120
AGGR = 5248
E2 = 86016
NW = 32
STEPS = 42
BLK = 64
ARS = AGGR // 16

_PACKS = {
    1: [(1, 2), (3, 4), (5,)],
    2: [(2, 3), (4, 5)],
    3: [(3, 4), (5,)],
    4: [(4, 5)],
    5: [(5,)],
}


def _mm1_body(a_ref, w_ref, b_ref, *out_refs):
    a = a_ref[...].astype(jnp.bfloat16)
    for i, o in enumerate(out_refs):
        o[...] = (
            jnp.dot(a, w_ref[i], preferred_element_type=jnp.float32) + b_ref[i]
        )


def _mm1(ea, w3, b3):
    R = 1280
    return pl.pallas_call(
        _mm1_body,
        grid=(E // R,),
        in_specs=[
            pl.BlockSpec((R, 345), lambda i: (i, 0)),
            pl.BlockSpec((3, 345, F), lambda i: (0, 0, 0)),
            pl.BlockSpec((3, F), lambda i: (0, 0)),
        ],
        out_specs=[pl.BlockSpec((R, F), lambda i: (i, 0))] * 3,
        out_shape=[jax.ShapeDtypeStruct((E, F), jnp.float32)] * 3,
    )(ea, w3, b3)


def _mm2_pair_body(off_l, off_r, el_ref, er_ref, wl_ref, wr_ref,
                   bl_ref, br_ref, o_ref):
    cl = jnp.dot(el_ref[...][:, off_l:off_l + 64], wl_ref[...],
                 preferred_element_type=jnp.float32) + bl_ref[...]
    cr = jnp.dot(er_ref[...][:, off_r:off_r + 64], wr_ref[...],
                 preferred_element_type=jnp.float32) + br_ref[...]
    o_ref[...] = jnp.concatenate([cl, cr], axis=1)


def _mm2_pair(el, er, wl, wr, bl, br, off_l, off_r):
    R = 2048
    return pl.pallas_call(
        functools.partial(_mm2_pair_body, off_l, off_r),
        grid=(E2 // R,),
        in_specs=[
            pl.BlockSpec((R, F), lambda i: (i, 0)),
            pl.BlockSpec((R, F), lambda i: (i, 0)),
            pl.BlockSpec((64, 64), lambda i: (0, 0)),
            pl.BlockSpec((64, 64), lambda i: (0, 0)),
            pl.BlockSpec((1, 64), lambda i: (0, 0)),
            pl.BlockSpec((1, 64), lambda i: (0, 0)),
        ],
        out_specs=pl.BlockSpec((R, F), lambda i: (i, 0)),
        out_shape=jax.ShapeDtypeStruct((E2, F), jnp.float32),
    )(el, er, wl, wr, bl, br)


def _mm2_single_body(off_l, el_ref, wl_ref, bl_ref, o_ref):
    cl = jnp.dot(el_ref[...][:, off_l:off_l + 64], wl_ref[...],
                 preferred_element_type=jnp.float32) + bl_ref[...]
    o_ref[...] = jnp.concatenate([cl, jnp.zeros_like(cl)], axis=1)


def _mm2_single(el, wl, bl, off_l):
    R = 2048
    return pl.pallas_call(
        functools.partial(_mm2_single_body, off_l),
        grid=(E2 // R,),
        in_specs=[
            pl.BlockSpec((R, F), lambda i: (i, 0)),
            pl.BlockSpec((64, 64), lambda i: (0, 0)),
            pl.BlockSpec((1, 64), lambda i: (0, 0)),
        ],
        out_specs=pl.BlockSpec((R, F), lambda i: (i, 0)),
        out_shape=jax.ShapeDtypeStruct((E2, F), jnp.float32),
    )(el, wl, bl)


def _proj_pair_body(hl_ref, hr_ref, wsl_ref, wdl_ref, wsr_ref, wdr_ref,
                    a_out, b_out):
    hl = hl_ref[...]
    hr = hr_ref[...]
    a_out[...] = jnp.zeros((NPAD, F), jnp.float32)
    b_out[...] = jnp.zeros((NPAD, F), jnp.float32)
    a_out[:N, :] = jnp.concatenate(
        [jnp.dot(hl, wsl_ref[...], preferred_element_type=jnp.float32),
         jnp.dot(hr, wsr_ref[...], preferred_element_type=jnp.float32)], axis=1)
    b_out[:N, :] = jnp.concatenate(
        [jnp.dot(hl, wdl_ref[...], preferred_element_type=jnp.float32),
         jnp.dot(hr, wdr_ref[...], preferred_element_type=jnp.float32)], axis=1)


def _proj_pair(hl, hr, wsl, wdl, wsr, wdr):
    return pl.pallas_call(
        _proj_pair_body,
        out_shape=[jax.ShapeDtypeStruct((NPAD, F), jnp.float32)] * 2,
    )(hl, hr, wsl, wdl, wsr, wdr)


def _proj_single_body(hl_ref, wsl_ref, wdl_ref, a_out, b_out):
    hl = hl_ref[...]
    a_out[...] = jnp.zeros((NPAD, F), jnp.float32)
    b_out[...] = jnp.zeros((NPAD, F), jnp.float32)
    al = jnp.dot(hl, wsl_ref[...], preferred_element_type=jnp.float32)
    bl = jnp.dot(hl, wdl_ref[...], preferred_element_type=jnp.float32)
    a_out[:N, :] = jnp.concatenate([al, jnp.zeros_like(al)], axis=1)
    b_out[:N, :] = jnp.concatenate([bl, jnp.zeros_like(bl)], axis=1)


def _proj_single(hl, wsl, wdl):
    return pl.pallas_call(
        _proj_single_body,
        out_shape=[jax.ShapeDtypeStruct((NPAD, F), jnp.float32)] * 2,
    )(hl, wsl, wdl)


def _nd_body(off, h_ref, wn_ref, bn_ref, aga_ref, agb_ref, g_ref, bt_ref,
             h_out):
    y = jnp.dot(h_ref[...], wn_ref[...], preferred_element_type=jnp.float32)
    agg = jnp.concatenate(
        [aga_ref[0, :H, off:off + 50] + aga_ref[1, :H, off:off + 50],
         agb_ref[0, :N - H, off:off + 50] + agb_ref[1, :N - H, off:off + 50]],
        axis=0)
    y = y + bn_ref[...] + agg
    mu = jnp.mean(y, axis=0, keepdims=True)
    var = jnp.mean((y - mu) ** 2, axis=0, keepdims=True)
    yn = g_ref[...] * (y - mu) * jax.lax.rsqrt(var + 1e-5) + bt_ref[...]
    h_out[...] = jnp.where(yn > 0, yn, jnp.exp(jnp.minimum(yn, 0.0)) - 1.0)


def _node_upd(h, wn, bn, aga, agb, g, bt, off):
    return pl.pallas_call(
        functools.partial(_nd_body, off),
        out_shape=jax.ShapeDtypeStruct((N, 50), jnp.float32),
    )(h, wn, bn, aga, agb, g, bt)


def _head_body(o_ref, w_ref, b_ref, g_ref, bt_ref, out_ref):
    y = jnp.dot(o_ref[...], w_ref[...], preferred_element_type=jnp.float32)
    y = y + b_ref[...]
    mu = jnp.mean(y, axis=0, keepdims=True)
    var = jnp.mean((y - mu) ** 2, axis=0, keepdims=True)
    out_ref[...] = g_ref[...] * (y - mu) * jax.lax.rsqrt(var + 1e-5) + bt_ref[...]


def _head(o, w, b, g, bt):
    return pl.pallas_call(
        _head_body,
        out_shape=jax.ShapeDtypeStruct((N, 196), jnp.float32),
    )(o, w, b, g, bt)


def _sc_body(layer1, a_hbm, b_hbm, c_hbm, si_hbm, di_hbm, dl_hbm, *rest):
    if layer1:
        (ci_hbm, eout_hbm, agg_hbm, idx_s, idx_d, idx_l, idx_c,
         buf_a, buf_b, buf_c, agg_sh, sem0, sem1, sem_c) = rest
    else:
        (eout_hbm, agg_hbm, idx_s, idx_d, idx_l, idx_c,
         buf_a, buf_b, buf_c, agg_sh, sem0, sem1, sem_c) = rest
        ci_hbm = None
    sems = (sem0, sem1)
    cid = lax.axis_index("c")
    sid = lax.axis_index("s")
    wid = cid * 16 + sid

    def zrow(r, c2):
        for q in range(F // 16):
            buf_a[0, r, pl.ds(q * 16, 16)] = jnp.zeros((16,), jnp.float32)
        return c2
    lax.fori_loop(0, BLK, zrow, 0)
    for k in range(ARS // BLK):
        pltpu.sync_copy(buf_a.at[0],
                        agg_sh.at[pl.ds(sid * ARS + k * BLK, BLK)])
    pltpu.sync_copy(buf_a.at[0, pl.ds(0, ARS - (ARS // BLK) * BLK)],
                    agg_sh.at[pl.ds(sid * ARS + (ARS // BLK) * BLK,
                                    ARS - (ARS // BLK) * BLK)])
    pltpu.sync_copy(si_hbm.at[wid], idx_s)
    pltpu.sync_copy(di_hbm.at[wid], idx_d)
    pltpu.sync_copy(dl_hbm.at[wid], idx_l)
    if layer1:
        pltpu.sync_copy(ci_hbm.at[wid], idx_c)
    plsc.subcore_barrier()

    def issue_ab(j, s):
        pltpu.async_copy(a_hbm.at[idx_s.at[j]], buf_a.at[s], sems[s])
        pltpu.async_copy(b_hbm.at[idx_d.at[j]], buf_b.at[s], sems[s])

    def wait_ab(j, s):
        pltpu.make_async_copy(a_hbm.at[idx_s.at[j]], buf_a.at[s],
                              sems[s]).wait()
        pltpu.make_async_copy(b_hbm.at[idx_d.at[j]], buf_b.at[s],
                              sems[s]).wait()

    issue_ab(0, 0)
    last = STEPS - 1

    def pair(p, carry):
        for r in (0, 1):
            j = 2 * p + r
            base = wid * (STEPS * BLK) + j * BLK
            issue_ab(jnp.minimum(j + 1, last), 1 - r)
            if layer1:
                cp_c = pltpu.async_copy(c_hbm.at[idx_c.at[j]], buf_c, sem_c)
                cp_c.wait()
            else:
                pltpu.sync_copy(c_hbm.at[pl.ds(base, BLK)], buf_c)
            wait_ab(j, r)

            def row(rr, c2):
                for q in range(F // 16):
                    v = (buf_a[r, rr, pl.ds(q * 16, 16)]
                         + buf_b[r, rr, pl.ds(q * 16, 16)]
                         + buf_c[rr, pl.ds(q * 16, 16)])
                    buf_c[rr, pl.ds(q * 16, 16)] = jnp.maximum(v, 0.0)
                return c2

            lax.fori_loop(0, BLK, row, 0)
            pltpu.sync_copy(buf_c, eout_hbm.at[pl.ds(base, BLK)])
            pltpu.sync_copy(buf_c, agg_sh.at[idx_l.at[j]], add=True)
        return carry

    lax.fori_loop(0, STEPS // 2, pair, 0)
    wait_ab(last, 0)
    plsc.subcore_barrier()
    pltpu.sync_copy(agg_sh.at[pl.ds(sid * ARS, ARS)],
                    agg_hbm.at[cid, pl.ds(sid * ARS, ARS)])


def _make_sc(layer1):
    scratch = [
        pltpu.VMEM((STEPS, BLK), jnp.int32),
        pltpu.VMEM((STEPS, BLK), jnp.int32),
        pltpu.VMEM((STEPS, BLK), jnp.int32),
        pltpu.VMEM((STEPS, BLK), jnp.int32),
        pltpu.VMEM((2, BLK, F), jnp.float32),
        pltpu.VMEM((2, BLK, F), jnp.float32),
        pltpu.VMEM((BLK, F), jnp.float32),
        pltpu.VMEM_SHARED((AGGR, F), jnp.float32),
        pltpu.SemaphoreType.DMA,
        pltpu.SemaphoreType.DMA,
        pltpu.SemaphoreType.DMA,
    ]
    return pl.kernel(
        functools.partial(_sc_body, layer1),
        out_type=[
            jax.ShapeDtypeStruct((E2, F), jnp.float32),
            jax.ShapeDtypeStruct((2, AGGR, F), jnp.float32),
        ],
        mesh=plsc.VectorSubcoreMesh(core_axis_name="c", subcore_axis_name="s"),
        scratch_types=scratch,
    )


_SC_CONV1 = _make_sc(True)
_SC_CONVN = _make_sc(False)


def _pad_w(w, rows, cols):
    out = jnp.zeros((rows, cols), jnp.float32)
    return out.at[: w.shape[0], : w.shape[1]].set(w)


def _partition(src, dst):
    eid = jnp.arange(E, dtype=jnp.int32)
    srcx = jnp.concatenate([src, jnp.array([N], jnp.int32)])
    dstx = jnp.concatenate([dst, jnp.array([N], jnp.int32)])
    halves = []
    for h in range(2):
        mask = (dst >= h * H) & (dst < (h + 1) * H)
        pos = jnp.where(mask, jnp.cumsum(mask) - 1, E2)
        slots = jnp.full((E2,), E, jnp.int32).at[pos].set(eid, mode="drop")
        s = srcx[slots]
        d = dstx[slots]
        dl = jnp.clip(d - h * H, 0, AGGR - 2)
        cidx = jnp.minimum(slots, E - 1)
        halves.append(tuple(a.reshape(NW, STEPS, BLK) for a in (s, d, dl, cidx)))
    return halves


def kernel(x, edge_index, edge_attr, condition, params):
    xc = jnp.concatenate([x, condition], axis=1)
    src = edge_index[0]
    dst = edge_index[1]
    halves = _partition(src, dst)

    def w1_half(b):
        return _pad_w(params["conv%d_1" % b]["We"][92:], 345, 64)

    def b1_half(b):
        return _pad_w(params["conv%d_1" % b]["be"][None, :], 1, 64)[0]

    zw = jnp.zeros((345, 64), jnp.float32)
    zb = jnp.zeros((64,), jnp.float32)
    w3 = jnp.stack([
        jnp.concatenate([w1_half(1), w1_half(2)], axis=1),
        jnp.concatenate([w1_half(3), w1_half(4)], axis=1),
        jnp.concatenate([w1_half(5), zw], axis=1),
    ]).astype(jnp.bfloat16)
    b3 = jnp.stack([
        jnp.concatenate([b1_half(1), b1_half(2)]),
        jnp.concatenate([b1_half(3), b1_half(4)]),
        jnp.concatenate([b1_half(5), zb]),
    ])
    c1 = _mm1(edge_attr, w3, b3)

    h = {b: xc for b in range(1, 6)}
    esrc = {}
    for pos in range(1, 6):
        for ip, pack in enumerate(_PACKS[pos]):
            convs = [params["conv%d_%d" % (b, pos)] for b in pack]
            in_c = convs[0]["Wn"].shape[0]
            ws = [_pad_w(p["We"][:in_c], in_c, 64) for p in convs]
            wd = [_pad_w(p["We"][in_c:2 * in_c], in_c, 64) for p in convs]
            if len(pack) == 2:
                a_nodes, b_nodes = _proj_pair(
                    h[pack[0]], h[pack[1]], ws[0], wd[0], ws[1], wd[1])
            else:
                a_nodes, b_nodes = _proj_single(h[pack[0]], ws[0], wd[0])
            if pos > 1:
                we = [_pad_w(p["We"][2 * in_c:], 64, 64) for p in convs]
                be = [_pad_w(p["be"][None, :], 1, 64) for p in convs]
            eo, ag = [], []
            for hi in range(2):
                si, di, dl, ci = halves[hi]
                if pos == 1:
                    e_h, agg_h = _SC_CONV1(a_nodes, b_nodes, c1[ip],
                                           si, di, dl, ci)
                else:
                    if len(pack) == 2:
                        (el_ab, offl) = esrc[pack[0]]
                        (er_ab, offr) = esrc[pack[1]]
                        c_h = _mm2_pair(el_ab[hi], er_ab[hi], we[0], we[1],
                                        be[0], be[1], offl, offr)
                    else:
                        (el_ab, offl) = esrc[pack[0]]
                        c_h = _mm2_single(el_ab[hi], we[0], be[0], offl)
                    e_h, agg_h = _SC_CONVN(a_nodes, b_nodes, c_h, si, di, dl)
                eo.append(e_h)
                ag.append(agg_h)
            for k, b in enumerate(pack):
                p = convs[k]
                bnp = params["BN%d_%d" % (b, pos)]
                h[b] = _node_upd(h[b], p["Wn"], p["bn"][None, :],
                                 ag[0], ag[1],
                                 bnp["gamma"][None, :], bnp["beta"][None, :],
                                 64 * k)
                esrc[b] = ((eo[0], eo[1]), 64 * k)

    o = jnp.concatenate([h[b] for b in range(1, 6)], axis=1)
    o2 = _head(
        o, params["linR_W"], params["linR_b"][None, :],
        params["BNR"]["gamma"][None, :], params["BNR"]["beta"][None, :],
    )
    return o2.reshape(-1, 49, 4)
